# baseline (speedup 1.0000x reference)
import jax
import jax.numpy as jnp
from jax.experimental import pallas as pl
from jax.experimental.pallas import tpu as pltpu

T = 2048
D = 1024
H = 2048
C = 1024
E = 8
K = 2
TB = 512
NT = T // TB


def _moe_dense_kernel(x_ref, Wg_ref, bg_ref, W1_ref, b1_ref, W2_ref, b2_ref,
                      out_ref, gates_ref, aux_ref,
                      acc_s, gates_s, w0_s, w1_s, i0_s, i1_s, fp_s):
    e = pl.program_id(0)
    t = pl.program_id(1)
    tok = pl.ds(t * TB, TB)

    @pl.when(e == 0)
    def _router():
        x = x_ref[...]
        logits = jax.lax.dot_general(
            x, Wg_ref[...], (((1,), (0,)), ((), ())),
            preferred_element_type=jnp.float32,
        ) + bg_ref[...]
        m = jnp.max(logits, axis=-1, keepdims=True)
        ex = jnp.exp(logits - m)
        gates = ex / jnp.sum(ex, axis=-1, keepdims=True)
        gates_s[tok, :] = gates

        cols = jax.lax.broadcasted_iota(jnp.int32, (TB, E), 1)
        m0 = jnp.max(gates, axis=-1, keepdims=True)
        i0 = jnp.min(jnp.where(gates == m0, cols, E), axis=-1, keepdims=True)
        g_masked = jnp.where(cols == i0, -jnp.inf, gates)
        m1 = jnp.max(g_masked, axis=-1, keepdims=True)
        i1 = jnp.min(jnp.where(g_masked == m1, cols, E), axis=-1, keepdims=True)
        denom = m0 + m1 + 1e-8
        w0_s[tok, :] = m0 / denom
        w1_s[tok, :] = m1 / denom
        i0_s[tok, :] = i0
        i1_s[tok, :] = i1

        mask = jnp.logical_or(cols == i0, cols == i1).astype(jnp.float32)
        fsum = jnp.sum(mask, axis=0, keepdims=True)
        psum = jnp.sum(gates, axis=0, keepdims=True)

        @pl.when(t == 0)
        def _():
            fp_s[0:1, :] = fsum
            fp_s[1:2, :] = psum

        @pl.when(t > 0)
        def _():
            fp_s[0:1, :] += fsum
            fp_s[1:2, :] += psum

    xb = x_ref[...].astype(jnp.bfloat16)
    h = jax.lax.dot_general(
        xb, W1_ref[...], (((1,), (0,)), ((), ())),
        preferred_element_type=jnp.float32,
    ) + b1_ref[...]
    h = jnp.maximum(h, 0.0).astype(jnp.bfloat16)
    y = jax.lax.dot_general(
        h, W2_ref[...], (((1,), (0,)), ((), ())),
        preferred_element_type=jnp.float32,
    ) + b2_ref[...]

    scale = (jnp.where(i0_s[tok, :] == e, w0_s[tok, :], 0.0)
             + jnp.where(i1_s[tok, :] == e, w1_s[tok, :], 0.0))

    @pl.when(e == 0)
    def _init():
        acc_s[tok, :] = scale * y

    @pl.when(e > 0)
    def _acc():
        acc_s[tok, :] += scale * y

    @pl.when(e == E - 1)
    def _out():
        out_ref[...] = acc_s[tok, :]
        gates_ref[...] = gates_s[tok, :]

    @pl.when(jnp.logical_and(e == E - 1, t == NT - 1))
    def _fin():
        aux_ref[0, :] = (jnp.sum(fp_s[0:1, :] * fp_s[1:2, :],
                                 axis=1, keepdims=True)
                         * (E / (T * T)))[0, :]


def kernel(x, Wg, bg, W1, b1, W2, b2):
    out, gates, aux = pl.pallas_call(
        _moe_dense_kernel,
        grid=(E, NT),
        in_specs=[
            pl.BlockSpec((TB, D), lambda e, t: (t, 0)),
            pl.BlockSpec((D, E), lambda e, t: (0, 0)),
            pl.BlockSpec((1, E), lambda e, t: (0, 0)),
            pl.BlockSpec((None, D, H), lambda e, t: (e, 0, 0)),
            pl.BlockSpec((None, 1, H), lambda e, t: (e, 0, 0)),
            pl.BlockSpec((None, H, C), lambda e, t: (e, 0, 0)),
            pl.BlockSpec((None, 1, C), lambda e, t: (e, 0, 0)),
        ],
        out_specs=[
            pl.BlockSpec((TB, C), lambda e, t: (t, 0)),
            pl.BlockSpec((TB, E), lambda e, t: (t, 0)),
            pl.BlockSpec((1, 1), lambda e, t: (0, 0)),
        ],
        out_shape=[
            jax.ShapeDtypeStruct((T, C), jnp.float32),
            jax.ShapeDtypeStruct((T, E), jnp.float32),
            jax.ShapeDtypeStruct((1, 1), jnp.float32),
        ],
        scratch_shapes=[
            pltpu.VMEM((T, C), jnp.float32),
            pltpu.VMEM((T, E), jnp.float32),
            pltpu.VMEM((T, 1), jnp.float32),
            pltpu.VMEM((T, 1), jnp.float32),
            pltpu.VMEM((T, 1), jnp.int32),
            pltpu.VMEM((T, 1), jnp.int32),
            pltpu.VMEM((2, E), jnp.float32),
        ],
    )(x, Wg, bg[None, :], W1.astype(jnp.bfloat16), b1[:, None, :],
      W2.astype(jnp.bfloat16), b2[:, None, :])
    return out, aux[0, 0], gates


# speedup vs baseline: 1.1967x; 1.1967x over previous
import functools
import jax
import jax.numpy as jnp
from jax import lax
from jax.experimental import pallas as pl
from jax.experimental.pallas import tpu as pltpu
from jax.experimental.pallas import tpu_sc as plsc

T = 2048
D = 1024
H = 2048
C = 1024
E = 8
K = 2

B = 256
G = (K * T) // B + E - 1
S = G * B

NW = 32
TPW = T // NW
CHUNK = 128
NCH = (K * T) // CHUNK


def _router_kernel(x_ref, Wg_ref, bg_ref,
                   gates_ref, aux_ref, dst_ref, wpair_ref, meta_ref):
    x = x_ref[...]
    logits = jax.lax.dot_general(
        x, Wg_ref[...], (((1,), (0,)), ((), ())),
        preferred_element_type=jnp.float32,
    ) + bg_ref[...]
    m = jnp.max(logits, axis=-1, keepdims=True)
    ex = jnp.exp(logits - m)
    gates = ex / jnp.sum(ex, axis=-1, keepdims=True)
    gates_ref[...] = gates

    cols = jax.lax.broadcasted_iota(jnp.int32, (T, E), 1)
    m0 = jnp.max(gates, axis=-1, keepdims=True)
    i0 = jnp.min(jnp.where(gates == m0, cols, E), axis=-1, keepdims=True)
    g_masked = jnp.where(cols == i0, -jnp.inf, gates)
    m1 = jnp.max(g_masked, axis=-1, keepdims=True)
    i1 = jnp.min(jnp.where(g_masked == m1, cols, E), axis=-1, keepdims=True)
    denom = m0 + m1 + 1e-8
    wpair_ref[...] = jnp.concatenate([m0 / denom, m1 / denom], axis=1)

    oh0 = (cols == i0).astype(jnp.float32)
    oh1 = (cols == i1).astype(jnp.float32)
    ohsum = oh0 + oh1

    ri = jax.lax.broadcasted_iota(jnp.int32, (CHUNK, CHUNK), 0)
    ci = jax.lax.broadcasted_iota(jnp.int32, (CHUNK, CHUNK), 1)
    L = (ci <= ri).astype(jnp.float32)

    nchunks = T // CHUNK
    excl_parts = []
    offs = jnp.zeros((1, E), dtype=jnp.float32)
    for c in range(nchunks):
        blk = ohsum[c * CHUNK:(c + 1) * CHUNK, :]
        incl = jax.lax.dot_general(
            L, blk, (((1,), (0,)), ((), ())),
            preferred_element_type=jnp.float32)
        excl_parts.append(incl - blk + offs)
        offs = offs + incl[CHUNK - 1:CHUNK, :]
    cum = jnp.concatenate(excl_parts, axis=0)
    counts = offs

    nb = jnp.ceil(counts * (1.0 / B))
    lowtri8 = (jax.lax.broadcasted_iota(jnp.int32, (E, E), 0)
               < jax.lax.broadcasted_iota(jnp.int32, (E, E), 1)).astype(jnp.float32)
    cnb_excl = jax.lax.dot_general(
        nb, lowtri8, (((1,), (0,)), ((), ())),
        preferred_element_type=jnp.float32)
    gpo = cnb_excl * float(B)
    slot = gpo + cum
    d0 = jnp.sum(oh0 * slot, axis=1, keepdims=True)
    d1 = jnp.sum(oh1 * slot, axis=1, keepdims=True)
    dst_ref[...] = jnp.concatenate([d0, d1], axis=1).astype(jnp.int32)

    cnb_incl = cnb_excl + nb
    tb = jnp.sum(nb)
    giota = jax.lax.broadcasted_iota(jnp.int32, (G, 1), 0).astype(jnp.float32)
    be = jnp.sum((cnb_incl <= giota).astype(jnp.float32),
                 axis=1, keepdims=True)
    be = jnp.minimum(be, float(E - 1))
    valid = (giota < tb).astype(jnp.float32)
    meta_ref[...] = jnp.concatenate([be, valid], axis=1).astype(jnp.int32)

    f = counts * (1.0 / T)
    P = jnp.sum(gates, axis=0, keepdims=True) * (1.0 / T)
    aux_ref[...] = E * jnp.sum(f * P, axis=1, keepdims=True)


def _router(x, Wg, bg):
    return pl.pallas_call(
        _router_kernel,
        out_shape=[
            jax.ShapeDtypeStruct((T, E), jnp.float32),
            jax.ShapeDtypeStruct((1, 1), jnp.float32),
            jax.ShapeDtypeStruct((T, K), jnp.int32),
            jax.ShapeDtypeStruct((T, K), jnp.float32),
            jax.ShapeDtypeStruct((G, 2), jnp.int32),
        ],
    )(x, Wg, bg[None, :])


def _dispatch_body(x_hbm, dst_hbm, xs_hbm, idx_v, rows_v, sem):
    wid = lax.axis_index("s") * 2 + lax.axis_index("c")
    pltpu.sync_copy(dst_hbm.at[wid], idx_v)
    pltpu.sync_copy(x_hbm.at[pl.ds(wid * TPW, TPW)], rows_v)
    pltpu.async_copy(rows_v, xs_hbm.at[idx_v.at[0]], sem).wait()
    pltpu.async_copy(rows_v, xs_hbm.at[idx_v.at[1]], sem).wait()


def _dispatch(x, dst_sc):
    mesh = plsc.VectorSubcoreMesh(core_axis_name="c", subcore_axis_name="s")
    return pl.kernel(
        _dispatch_body,
        out_type=jax.ShapeDtypeStruct((S, D), jnp.float32),
        mesh=mesh,
        scratch_types=[
            pltpu.VMEM((K, TPW), jnp.int32),
            pltpu.VMEM((TPW, D), jnp.float32),
            pltpu.SemaphoreType.DMA,
        ],
    )(x, dst_sc)


def _gemm_kernel(be_ref, valid_ref, xs_ref, W1_ref, b1_ref, W2_ref, b2_ref,
                 out_ref):
    g = pl.program_id(0)

    @pl.when(valid_ref[g] == 1)
    def _():
        xb = xs_ref[...].astype(jnp.bfloat16)
        h = jax.lax.dot_general(
            xb, W1_ref[...], (((1,), (0,)), ((), ())),
            preferred_element_type=jnp.float32,
        ) + b1_ref[...]
        hb = jnp.maximum(h, 0.0).astype(jnp.bfloat16)
        y = jax.lax.dot_general(
            hb, W2_ref[...], (((1,), (0,)), ((), ())),
            preferred_element_type=jnp.float32,
        ) + b2_ref[...]
        out_ref[...] = y


def _grouped_gemm(xs, W1b, b1r, W2b, b2r, be, valid):
    grid_spec = pltpu.PrefetchScalarGridSpec(
        num_scalar_prefetch=2,
        grid=(G,),
        in_specs=[
            pl.BlockSpec((B, D), lambda g, be, va: (g, 0)),
            pl.BlockSpec((None, D, H), lambda g, be, va: (be[g], 0, 0)),
            pl.BlockSpec((None, 1, H), lambda g, be, va: (be[g], 0, 0)),
            pl.BlockSpec((None, H, C), lambda g, be, va: (be[g], 0, 0)),
            pl.BlockSpec((None, 1, C), lambda g, be, va: (be[g], 0, 0)),
        ],
        out_specs=pl.BlockSpec((B, C), lambda g, be, va: (g, 0)),
    )
    return pl.pallas_call(
        _gemm_kernel,
        grid_spec=grid_spec,
        out_shape=jax.ShapeDtypeStruct((S, C), jnp.float32),
    )(be, valid, xs, W1b, b1r, W2b, b2r)


def _combine_body(ys_hbm, dst_hbm, wp_hbm, out_hbm,
                  idx_v, wp_v, y0_v, y1_v, o_v, sem0, sem1):
    wid = lax.axis_index("s") * 2 + lax.axis_index("c")
    pltpu.sync_copy(dst_hbm.at[wid], idx_v)
    pltpu.sync_copy(wp_hbm.at[wid], wp_v)
    half = TPW // 2
    for cc in range(2):
        j0 = cc * half
        cp0 = pltpu.async_copy(ys_hbm.at[idx_v.at[0, pl.ds(j0, half)]],
                               y0_v, sem0)
        cp1 = pltpu.async_copy(ys_hbm.at[idx_v.at[1, pl.ds(j0, half)]],
                               y1_v, sem1)
        cp0.wait()
        cp1.wait()

        w0s = []
        w1s = []
        for q in range(half // 16):
            w0vec = wp_v[0, pl.ds(j0 + q * 16, 16)]
            w1vec = wp_v[1, pl.ds(j0 + q * 16, 16)]
            for i in range(16):
                w0s.append(w0vec[i])
                w1s.append(w1vec[i])

        def body(v, _):
            sl = pl.ds(v * 16, 16)
            for j in range(half):
                o_v[j, sl] = y0_v[j, sl] * w0s[j] + y1_v[j, sl] * w1s[j]
            return 0

        lax.fori_loop(0, C // 16, body, 0)
        pltpu.sync_copy(o_v, out_hbm.at[pl.ds(wid * TPW + j0, half)])


def _combine(ys, dst_sc, wp_sc):
    mesh = plsc.VectorSubcoreMesh(core_axis_name="c", subcore_axis_name="s")
    half = TPW // 2
    return pl.kernel(
        _combine_body,
        out_type=jax.ShapeDtypeStruct((T, C), jnp.float32),
        mesh=mesh,
        scratch_types=[
            pltpu.VMEM((K, TPW), jnp.int32),
            pltpu.VMEM((K, TPW), jnp.float32),
            pltpu.VMEM((half, C), jnp.float32),
            pltpu.VMEM((half, C), jnp.float32),
            pltpu.VMEM((half, C), jnp.float32),
            pltpu.SemaphoreType.DMA,
            pltpu.SemaphoreType.DMA,
        ],
    )(ys, dst_sc, wp_sc)


def kernel(x, Wg, bg, W1, b1, W2, b2):
    gates, aux, dst, wpair, meta = _router(x, Wg, bg)

    dst_sc = dst.reshape(NW, TPW, K).transpose(0, 2, 1)
    wp_sc = wpair.reshape(NW, TPW, K).transpose(0, 2, 1)

    xs = _dispatch(x, dst_sc)

    ys = _grouped_gemm(
        xs,
        W1.astype(jnp.bfloat16), b1[:, None, :],
        W2.astype(jnp.bfloat16), b2[:, None, :],
        meta[:, 0], meta[:, 1],
    )

    out = _combine(ys, dst_sc, wp_sc)
    return out, aux[0, 0], gates


# speedup vs baseline: 1.5132x; 1.2645x over previous
import functools
import jax
import jax.numpy as jnp
from jax import lax
from jax.experimental import pallas as pl
from jax.experimental.pallas import tpu as pltpu
from jax.experimental.pallas import tpu_sc as plsc

T = 2048
D = 1024
H = 2048
C = 1024
E = 8
K = 2

B = 256
G = (K * T) // B + E - 1
S = G * B

NW = 32
TPW = T // NW
CHUNK = 128
NCH = (K * T) // CHUNK


def _router_kernel(x_ref, Wg_ref, bg_ref,
                   gates_ref, aux_ref, dst_ref, wpair_ref, meta_ref):
    x = x_ref[...]
    logits = jax.lax.dot_general(
        x, Wg_ref[...], (((1,), (0,)), ((), ())),
        preferred_element_type=jnp.float32,
    ) + bg_ref[...]
    m = jnp.max(logits, axis=-1, keepdims=True)
    ex = jnp.exp(logits - m)
    gates = ex / jnp.sum(ex, axis=-1, keepdims=True)
    gates_ref[...] = gates

    cols = jax.lax.broadcasted_iota(jnp.int32, (T, E), 1)
    m0 = jnp.max(gates, axis=-1, keepdims=True)
    i0 = jnp.min(jnp.where(gates == m0, cols, E), axis=-1, keepdims=True)
    g_masked = jnp.where(cols == i0, -jnp.inf, gates)
    m1 = jnp.max(g_masked, axis=-1, keepdims=True)
    i1 = jnp.min(jnp.where(g_masked == m1, cols, E), axis=-1, keepdims=True)
    denom = m0 + m1 + 1e-8
    wpair_ref[...] = jnp.concatenate([m0 / denom, m1 / denom], axis=1)

    oh0 = (cols == i0).astype(jnp.float32)
    oh1 = (cols == i1).astype(jnp.float32)
    ohsum = oh0 + oh1

    ri = jax.lax.broadcasted_iota(jnp.int32, (CHUNK, CHUNK), 0)
    ci = jax.lax.broadcasted_iota(jnp.int32, (CHUNK, CHUNK), 1)
    L = (ci <= ri).astype(jnp.float32)

    nchunks = T // CHUNK
    excl_parts = []
    offs = jnp.zeros((1, E), dtype=jnp.float32)
    for c in range(nchunks):
        blk = ohsum[c * CHUNK:(c + 1) * CHUNK, :]
        incl = jax.lax.dot_general(
            L, blk, (((1,), (0,)), ((), ())),
            preferred_element_type=jnp.float32)
        excl_parts.append(incl - blk + offs)
        offs = offs + incl[CHUNK - 1:CHUNK, :]
    cum = jnp.concatenate(excl_parts, axis=0)
    counts = offs

    nb = jnp.ceil(counts * (1.0 / B))
    lowtri8 = (jax.lax.broadcasted_iota(jnp.int32, (E, E), 0)
               < jax.lax.broadcasted_iota(jnp.int32, (E, E), 1)).astype(jnp.float32)
    cnb_excl = jax.lax.dot_general(
        nb, lowtri8, (((1,), (0,)), ((), ())),
        preferred_element_type=jnp.float32)
    gpo = cnb_excl * float(B)
    slot = gpo + cum
    d0 = jnp.sum(oh0 * slot, axis=1, keepdims=True)
    d1 = jnp.sum(oh1 * slot, axis=1, keepdims=True)
    dst_ref[...] = jnp.concatenate([d0, d1], axis=1).astype(jnp.int32)

    cnb_incl = cnb_excl + nb
    tb = jnp.sum(nb)
    giota = jax.lax.broadcasted_iota(jnp.int32, (G, 1), 0).astype(jnp.float32)
    be = jnp.sum((cnb_incl <= giota).astype(jnp.float32),
                 axis=1, keepdims=True)
    be = jnp.minimum(be, float(E - 1))
    valid = (giota < tb).astype(jnp.float32)
    meta_ref[...] = jnp.concatenate([be, valid], axis=1).astype(jnp.int32)

    f = counts * (1.0 / T)
    P = jnp.sum(gates, axis=0, keepdims=True) * (1.0 / T)
    aux_ref[...] = E * jnp.sum(f * P, axis=1, keepdims=True)


def _router(x, Wg, bg):
    return pl.pallas_call(
        _router_kernel,
        out_shape=[
            jax.ShapeDtypeStruct((T, E), jnp.float32),
            jax.ShapeDtypeStruct((1, 1), jnp.float32),
            jax.ShapeDtypeStruct((T, K), jnp.int32),
            jax.ShapeDtypeStruct((T, K), jnp.float32),
            jax.ShapeDtypeStruct((G, 2), jnp.int32),
        ],
    )(x, Wg, bg[None, :])


def _dispatch_body(x_hbm, dst_hbm, xs_hbm, idx_v, rows_v, sem, sem2):
    wid = lax.axis_index("s") * 2 + lax.axis_index("c")
    pltpu.sync_copy(dst_hbm.at[wid], idx_v)
    pltpu.sync_copy(x_hbm.at[pl.ds(wid * TPW, TPW)], rows_v)
    cp0 = pltpu.async_copy(rows_v, xs_hbm.at[idx_v.at[0]], sem)
    cp1 = pltpu.async_copy(rows_v, xs_hbm.at[idx_v.at[1]], sem2)
    cp0.wait()
    cp1.wait()


def _dispatch(x, dst_sc):
    mesh = plsc.VectorSubcoreMesh(core_axis_name="c", subcore_axis_name="s")
    return pl.kernel(
        _dispatch_body,
        out_type=jax.ShapeDtypeStruct((S, D), jnp.float32),
        mesh=mesh,
        scratch_types=[
            pltpu.VMEM((K, TPW), jnp.int32),
            pltpu.VMEM((TPW, D), jnp.float32),
            pltpu.SemaphoreType.DMA,
            pltpu.SemaphoreType.DMA,
        ],
    )(x, dst_sc)


def _gemm_kernel(be_ref, valid_ref, xs_ref, W1_ref, b1_ref, W2_ref, b2_ref,
                 out_ref):
    g = pl.program_id(0)

    @pl.when(valid_ref[g] == 1)
    def _():
        h = jax.lax.dot_general(
            xs_ref[...], W1_ref[...], (((1,), (0,)), ((), ())),
            preferred_element_type=jnp.float32,
        ) + b1_ref[...]
        hb = jnp.maximum(h, 0.0)
        y = jax.lax.dot_general(
            hb, W2_ref[...], (((1,), (0,)), ((), ())),
            preferred_element_type=jnp.float32,
        ) + b2_ref[...]
        out_ref[...] = y


def _grouped_gemm(xs, W1b, b1r, W2b, b2r, be, valid):
    grid_spec = pltpu.PrefetchScalarGridSpec(
        num_scalar_prefetch=2,
        grid=(G,),
        in_specs=[
            pl.BlockSpec((B, D), lambda g, be, va: (g, 0)),
            pl.BlockSpec((None, D, H), lambda g, be, va: (be[g], 0, 0)),
            pl.BlockSpec((None, 1, H), lambda g, be, va: (be[g], 0, 0)),
            pl.BlockSpec((None, H, C), lambda g, be, va: (be[g], 0, 0)),
            pl.BlockSpec((None, 1, C), lambda g, be, va: (be[g], 0, 0)),
        ],
        out_specs=pl.BlockSpec((B, C), lambda g, be, va: (g, 0)),
    )
    return pl.pallas_call(
        _gemm_kernel,
        grid_spec=grid_spec,
        out_shape=jax.ShapeDtypeStruct((S, C), jnp.float32),
    )(be, valid, xs, W1b, b1r, W2b, b2r)


def _combine_body(ys_hbm, dst_hbm, wp_hbm, out_hbm,
                  idx_v, wp_v, y0_v, y1_v, o_v, sem0, sem1):
    wid = lax.axis_index("s") * 2 + lax.axis_index("c")
    pltpu.sync_copy(dst_hbm.at[wid], idx_v)
    pltpu.sync_copy(wp_hbm.at[wid], wp_v)
    half = TPW // 2
    for cc in range(2):
        j0 = cc * half
        cp0 = pltpu.async_copy(ys_hbm.at[idx_v.at[0, pl.ds(j0, half)]],
                               y0_v, sem0)
        cp1 = pltpu.async_copy(ys_hbm.at[idx_v.at[1, pl.ds(j0, half)]],
                               y1_v, sem1)
        cp0.wait()
        cp1.wait()

        w0s = []
        w1s = []
        for q in range(half // 16):
            w0vec = wp_v[0, pl.ds(j0 + q * 16, 16)]
            w1vec = wp_v[1, pl.ds(j0 + q * 16, 16)]
            for i in range(16):
                w0s.append(w0vec[i])
                w1s.append(w1vec[i])

        def body(v, _):
            sl = pl.ds(v * 16, 16)
            for j in range(half):
                o_v[j, sl] = y0_v[j, sl] * w0s[j] + y1_v[j, sl] * w1s[j]
            return 0

        lax.fori_loop(0, C // 16, body, 0)
        pltpu.sync_copy(o_v, out_hbm.at[pl.ds(wid * TPW + j0, half)])


def _combine(ys, dst_sc, wp_sc):
    mesh = plsc.VectorSubcoreMesh(core_axis_name="c", subcore_axis_name="s")
    half = TPW // 2
    return pl.kernel(
        _combine_body,
        out_type=jax.ShapeDtypeStruct((T, C), jnp.float32),
        mesh=mesh,
        scratch_types=[
            pltpu.VMEM((K, TPW), jnp.int32),
            pltpu.VMEM((K, TPW), jnp.float32),
            pltpu.VMEM((half, C), jnp.float32),
            pltpu.VMEM((half, C), jnp.float32),
            pltpu.VMEM((half, C), jnp.float32),
            pltpu.SemaphoreType.DMA,
            pltpu.SemaphoreType.DMA,
        ],
    )(ys, dst_sc, wp_sc)


def kernel(x, Wg, bg, W1, b1, W2, b2):
    gates, aux, dst, wpair, meta = _router(x, Wg, bg)

    dst_sc = dst.reshape(NW, TPW, K).transpose(0, 2, 1)
    wp_sc = wpair.reshape(NW, TPW, K).transpose(0, 2, 1)

    xs = _dispatch(x, dst_sc)

    ys = _grouped_gemm(
        xs,
        W1, b1[:, None, :],
        W2, b2[:, None, :],
        meta[:, 0], meta[:, 1],
    )

    out = _combine(ys, dst_sc, wp_sc)
    return out, aux[0, 0], gates


# speedup vs baseline: 1.5474x; 1.0226x over previous
import functools
import jax
import jax.numpy as jnp
from jax import lax
from jax.experimental import pallas as pl
from jax.experimental.pallas import tpu as pltpu
from jax.experimental.pallas import tpu_sc as plsc

T = 2048
D = 1024
H = 2048
C = 1024
E = 8
K = 2

B = 256
G = (K * T) // B + E - 1
S = G * B

NW = 32
TPW = T // NW
CHUNK = 128
NCH = (K * T) // CHUNK


def _router_kernel(x_ref, Wg_ref, bg_ref,
                   gates_ref, aux_ref, dst_ref, wpair_ref, meta_ref):
    x = x_ref[...]
    logits = jax.lax.dot_general(
        x, Wg_ref[...], (((1,), (0,)), ((), ())),
        preferred_element_type=jnp.float32,
    ) + bg_ref[...]
    m = jnp.max(logits, axis=-1, keepdims=True)
    ex = jnp.exp(logits - m)
    gates = ex / jnp.sum(ex, axis=-1, keepdims=True)
    gates_ref[...] = gates

    cols = jax.lax.broadcasted_iota(jnp.int32, (T, E), 1)
    m0 = jnp.max(gates, axis=-1, keepdims=True)
    i0 = jnp.min(jnp.where(gates == m0, cols, E), axis=-1, keepdims=True)
    g_masked = jnp.where(cols == i0, -jnp.inf, gates)
    m1 = jnp.max(g_masked, axis=-1, keepdims=True)
    i1 = jnp.min(jnp.where(g_masked == m1, cols, E), axis=-1, keepdims=True)
    denom = m0 + m1 + 1e-8
    w0 = m0 / denom
    w1 = m1 / denom
    wpair_ref[...] = jnp.concatenate(
        [jnp.reshape(w0, (NW, 1, TPW)), jnp.reshape(w1, (NW, 1, TPW))],
        axis=1)

    oh0 = (cols == i0).astype(jnp.float32)
    oh1 = (cols == i1).astype(jnp.float32)
    ohsum = oh0 + oh1

    ri = jax.lax.broadcasted_iota(jnp.int32, (CHUNK, CHUNK), 0)
    ci = jax.lax.broadcasted_iota(jnp.int32, (CHUNK, CHUNK), 1)
    L = (ci <= ri).astype(jnp.float32)

    nchunks = T // CHUNK
    excl_parts = []
    offs = jnp.zeros((1, E), dtype=jnp.float32)
    for c in range(nchunks):
        blk = ohsum[c * CHUNK:(c + 1) * CHUNK, :]
        incl = jax.lax.dot_general(
            L, blk, (((1,), (0,)), ((), ())),
            preferred_element_type=jnp.float32)
        excl_parts.append(incl - blk + offs)
        offs = offs + incl[CHUNK - 1:CHUNK, :]
    cum = jnp.concatenate(excl_parts, axis=0)
    counts = offs

    nb = jnp.ceil(counts * (1.0 / B))
    lowtri8 = (jax.lax.broadcasted_iota(jnp.int32, (E, E), 0)
               < jax.lax.broadcasted_iota(jnp.int32, (E, E), 1)).astype(jnp.float32)
    cnb_excl = jax.lax.dot_general(
        nb, lowtri8, (((1,), (0,)), ((), ())),
        preferred_element_type=jnp.float32)
    gpo = cnb_excl * float(B)
    slot = gpo + cum
    d0 = jnp.sum(oh0 * slot, axis=1, keepdims=True)
    d1 = jnp.sum(oh1 * slot, axis=1, keepdims=True)
    dst_ref[...] = jnp.concatenate(
        [jnp.reshape(d0, (NW, 1, TPW)), jnp.reshape(d1, (NW, 1, TPW))],
        axis=1).astype(jnp.int32)

    cnb_incl = cnb_excl + nb
    tb = jnp.sum(nb)
    giota = jax.lax.broadcasted_iota(jnp.int32, (G, 1), 0).astype(jnp.float32)
    be = jnp.sum((cnb_incl <= giota).astype(jnp.float32),
                 axis=1, keepdims=True)
    be = jnp.minimum(be, float(E - 1))
    valid = (giota < tb).astype(jnp.float32)
    meta_ref[...] = jnp.concatenate([be, valid], axis=1).astype(jnp.int32)

    f = counts * (1.0 / T)
    P = jnp.sum(gates, axis=0, keepdims=True) * (1.0 / T)
    aux_ref[...] = E * jnp.sum(f * P, axis=1, keepdims=True)


def _router(x, Wg, bg):
    return pl.pallas_call(
        _router_kernel,
        out_shape=[
            jax.ShapeDtypeStruct((T, E), jnp.float32),
            jax.ShapeDtypeStruct((1, 1), jnp.float32),
            jax.ShapeDtypeStruct((NW, K, TPW), jnp.int32),
            jax.ShapeDtypeStruct((NW, K, TPW), jnp.float32),
            jax.ShapeDtypeStruct((G, 2), jnp.int32),
        ],
    )(x, Wg, bg[None, :])


def _dispatch_body(x_hbm, dst_hbm, xs_hbm, idx_v, rows_v, sem, sem2):
    wid = lax.axis_index("s") * 2 + lax.axis_index("c")
    pltpu.sync_copy(dst_hbm.at[wid], idx_v)
    pltpu.sync_copy(x_hbm.at[pl.ds(wid * TPW, TPW)], rows_v)
    cp0 = pltpu.async_copy(rows_v, xs_hbm.at[idx_v.at[0]], sem)
    cp1 = pltpu.async_copy(rows_v, xs_hbm.at[idx_v.at[1]], sem2)
    cp0.wait()
    cp1.wait()


def _dispatch(x, dst):
    mesh = plsc.VectorSubcoreMesh(core_axis_name="c", subcore_axis_name="s")
    return pl.kernel(
        _dispatch_body,
        out_type=jax.ShapeDtypeStruct((S, D), jnp.float32),
        mesh=mesh,
        scratch_types=[
            pltpu.VMEM((K, TPW), jnp.int32),
            pltpu.VMEM((TPW, D), jnp.float32),
            pltpu.SemaphoreType.DMA,
            pltpu.SemaphoreType.DMA,
        ],
    )(x, dst)


def _gemm_kernel(be_ref, valid_ref, xs_ref, W1_ref, b1_ref, W2_ref, b2_ref,
                 out_ref):
    g = pl.program_id(0)

    @pl.when(valid_ref[g] == 1)
    def _():
        h = jax.lax.dot_general(
            xs_ref[...], W1_ref[...], (((1,), (0,)), ((), ())),
            preferred_element_type=jnp.float32,
        ) + b1_ref[...]
        hb = jnp.maximum(h, 0.0)
        y = jax.lax.dot_general(
            hb, W2_ref[...], (((1,), (0,)), ((), ())),
            preferred_element_type=jnp.float32,
        ) + b2_ref[...]
        out_ref[...] = y


def _grouped_gemm(xs, W1b, b1r, W2b, b2r, be, valid):
    grid_spec = pltpu.PrefetchScalarGridSpec(
        num_scalar_prefetch=2,
        grid=(G,),
        in_specs=[
            pl.BlockSpec((B, D), lambda g, be, va: (g, 0)),
            pl.BlockSpec((None, D, H), lambda g, be, va: (be[g], 0, 0)),
            pl.BlockSpec((None, 1, H), lambda g, be, va: (be[g], 0, 0)),
            pl.BlockSpec((None, H, C), lambda g, be, va: (be[g], 0, 0)),
            pl.BlockSpec((None, 1, C), lambda g, be, va: (be[g], 0, 0)),
        ],
        out_specs=pl.BlockSpec((B, C), lambda g, be, va: (g, 0)),
    )
    return pl.pallas_call(
        _gemm_kernel,
        grid_spec=grid_spec,
        out_shape=jax.ShapeDtypeStruct((S, C), jnp.float32),
    )(be, valid, xs, W1b, b1r, W2b, b2r)


def _combine_body(ys_hbm, dst_hbm, wp_hbm, out_hbm,
                  idx_v, wp_v, y0_v, y1_v, o_v, sem0, sem1):
    wid = lax.axis_index("s") * 2 + lax.axis_index("c")
    pltpu.sync_copy(dst_hbm.at[wid], idx_v)
    pltpu.sync_copy(wp_hbm.at[wid], wp_v)
    half = TPW // 2
    for cc in range(2):
        j0 = cc * half
        cp0 = pltpu.async_copy(ys_hbm.at[idx_v.at[0, pl.ds(j0, half)]],
                               y0_v, sem0)
        cp1 = pltpu.async_copy(ys_hbm.at[idx_v.at[1, pl.ds(j0, half)]],
                               y1_v, sem1)
        cp0.wait()
        cp1.wait()

        w0s = []
        w1s = []
        for q in range(half // 16):
            w0vec = wp_v[0, pl.ds(j0 + q * 16, 16)]
            w1vec = wp_v[1, pl.ds(j0 + q * 16, 16)]
            for i in range(16):
                w0s.append(w0vec[i])
                w1s.append(w1vec[i])

        def body(v, _):
            sl = pl.ds(v * 16, 16)
            for j in range(half):
                o_v[j, sl] = y0_v[j, sl] * w0s[j] + y1_v[j, sl] * w1s[j]
            return 0

        lax.fori_loop(0, C // 16, body, 0)
        pltpu.sync_copy(o_v, out_hbm.at[pl.ds(wid * TPW + j0, half)])


def _combine(ys, dst, wpair):
    mesh = plsc.VectorSubcoreMesh(core_axis_name="c", subcore_axis_name="s")
    half = TPW // 2
    return pl.kernel(
        _combine_body,
        out_type=jax.ShapeDtypeStruct((T, C), jnp.float32),
        mesh=mesh,
        scratch_types=[
            pltpu.VMEM((K, TPW), jnp.int32),
            pltpu.VMEM((K, TPW), jnp.float32),
            pltpu.VMEM((half, C), jnp.float32),
            pltpu.VMEM((half, C), jnp.float32),
            pltpu.VMEM((half, C), jnp.float32),
            pltpu.SemaphoreType.DMA,
            pltpu.SemaphoreType.DMA,
        ],
    )(ys, dst, wpair)


def kernel(x, Wg, bg, W1, b1, W2, b2):
    gates, aux, dst, wpair, meta = _router(x, Wg, bg)

    xs = _dispatch(x, dst)

    ys = _grouped_gemm(
        xs,
        W1, b1[:, None, :],
        W2, b2[:, None, :],
        meta[:, 0], meta[:, 1],
    )

    out = _combine(ys, dst, wpair)
    return out, aux[0, 0], gates
